# Initial kernel scaffold; baseline (speedup 1.0000x reference)
import jax
import jax.numpy as jnp
from jax.experimental import pallas as pl


def kernel(input_ids, W):
    raise NotImplementedError("write your pallas kernel here")


# speedup vs baseline: 1.1403x; 1.1403x over previous
import functools

import jax
import jax.numpy as jnp
from jax import lax
from jax.experimental import pallas as pl
from jax.experimental.pallas import tpu as pltpu
from jax.experimental.pallas import tpu_sc as plsc

B, S, D = 4, 4096, 2048
NC, NS, L = 2, 16, 16
NW = NC * NS
CHUNK = (B * S) // NW
CPR = S // CHUNK
G = 16
NG = CHUNK // G


_cummax = plsc.cummax


def _wid():
    return lax.axis_index("s") * NC + lax.axis_index("c")


def _body(ids_hbm, w_hbm, out_hbm, row_v, idx_v, buf_v, sem_g):
    wid = _wid()
    r = wid // CPR
    k = wid % CPR
    base = k * CHUNK

    pltpu.sync_copy(ids_hbm.at[r], row_v)

    lane = lax.iota(jnp.int32, 16)

    def prefix_step(j, carry):
        ids = row_v[pl.ds(j * L, L)]
        nd = jnp.where(ids < 10, jnp.int32(-1), lane + j * L)
        return jnp.maximum(carry, jnp.max(nd))

    carry0 = lax.fori_loop(0, base // L, prefix_step, jnp.int32(-1))

    def chunk_step(j, carry):
        off = base + j * L
        ids = row_v[pl.ds(off, L)]
        mask = ids < 10
        idxv = lane + off
        nd = jnp.where(mask, jnp.int32(-1), idxv)
        cm = jnp.maximum(_cummax(nd), lax.broadcast(carry, (L,)))
        pos = jnp.where(mask, idxv - cm, jnp.int32(0))
        idx_v[j] = pos
        return jnp.maximum(carry, jnp.max(nd))

    lax.fori_loop(0, CHUNK // L, chunk_step, carry0)

    out_base = wid * CHUNK

    def gather_step(j, c):
        pltpu.async_copy(w_hbm.at[idx_v[j]], buf_v, sem_g).wait()
        pltpu.sync_copy(buf_v, out_hbm.at[pl.ds(out_base + j * G, G)])
        return c

    lax.fori_loop(0, NG, gather_step, 0)


@jax.jit
def _run(input_ids, w):
    mesh = plsc.VectorSubcoreMesh(
        core_axis_name="c", subcore_axis_name="s", num_cores=NC, num_subcores=NS
    )
    f = pl.kernel(
        _body,
        out_type=jax.ShapeDtypeStruct((B * S, D), jnp.float32),
        mesh=mesh,
        scratch_types=[
            pltpu.VMEM((S,), jnp.int32),
            pltpu.VMEM((NG, G), jnp.int32),
            pltpu.VMEM((G, D), jnp.float32),
            pltpu.SemaphoreType.DMA,
        ],
        compiler_params=pltpu.CompilerParams(needs_layout_passes=False),
    )
    return f(input_ids, w).reshape(B, S, D)


def kernel(input_ids, W):
    return _run(input_ids, W)


# speedup vs baseline: 1.1407x; 1.0004x over previous
import functools

import jax
import jax.numpy as jnp
from jax import lax
from jax.experimental import pallas as pl
from jax.experimental.pallas import tpu as pltpu
from jax.experimental.pallas import tpu_sc as plsc

B, S, D = 4, 4096, 2048
NC, NS, L = 2, 16, 16
NW = NC * NS
CHUNK = (B * S) // NW
CPR = S // CHUNK
G = 16
NG = CHUNK // G


_cummax = plsc.cummax


def _wid():
    return lax.axis_index("s") * NC + lax.axis_index("c")


def _body(ids_hbm, w_hbm, out_hbm, row_v, idx_v, buf_v, sem_g, sem_s):
    wid = _wid()
    r = wid // CPR
    k = wid % CPR
    base = k * CHUNK

    pltpu.sync_copy(ids_hbm.at[r], row_v)

    lane = lax.iota(jnp.int32, 16)

    def prefix_step(j, carry):
        ids = row_v[pl.ds(j * L, L)]
        nd = jnp.where(ids < 10, jnp.int32(-1), lane + j * L)
        return jnp.maximum(carry, jnp.max(nd))

    carry0 = lax.fori_loop(0, base // L, prefix_step, jnp.int32(-1))

    def chunk_step(j, carry):
        off = base + j * L
        ids = row_v[pl.ds(off, L)]
        mask = ids < 10
        idxv = lane + off
        nd = jnp.where(mask, jnp.int32(-1), idxv)
        cm = jnp.maximum(_cummax(nd), lax.broadcast(carry, (L,)))
        pos = jnp.where(mask, idxv - cm, jnp.int32(0))
        idx_v[j] = pos
        return jnp.maximum(carry, jnp.max(nd))

    lax.fori_loop(0, CHUNK // L, chunk_step, carry0)

    out_base = wid * CHUNK

    def start_gather(j, b):
        pltpu.async_copy(w_hbm.at[idx_v[j]], buf_v.at[b], sem_g.at[b])

    def start_write(j, b):
        pltpu.async_copy(
            buf_v.at[b], out_hbm.at[pl.ds(out_base + j * G, G)], sem_s.at[b]
        )

    def wait_gather(j, b):
        pltpu.make_async_copy(w_hbm.at[idx_v[j]], buf_v.at[b], sem_g.at[b]).wait()

    def wait_write(j, b):
        pltpu.make_async_copy(
            buf_v.at[b], out_hbm.at[pl.ds(out_base + j * G, G)], sem_s.at[b]
        ).wait()

    start_gather(0, 0)

    def pipe_step(i, c):
        for b in range(2):
            j = 2 * i + b
            wait_gather(j, b)
            start_write(j, b)

            @pl.when(j + 1 < NG)
            def _():
                @pl.when(j >= 1)
                def _():
                    wait_write(j - 1, 1 - b)

                start_gather(j + 1, 1 - b)

        return c

    lax.fori_loop(0, NG // 2, pipe_step, 0)
    wait_write(NG - 2, 0)
    wait_write(NG - 1, 1)


@jax.jit
def _run(input_ids, w):
    mesh = plsc.VectorSubcoreMesh(
        core_axis_name="c", subcore_axis_name="s", num_cores=NC, num_subcores=NS
    )
    f = pl.kernel(
        _body,
        out_type=jax.ShapeDtypeStruct((B * S, D), jnp.float32),
        mesh=mesh,
        scratch_types=[
            pltpu.VMEM((S,), jnp.int32),
            pltpu.VMEM((NG, G), jnp.int32),
            pltpu.VMEM((2, G, D), jnp.float32),
            pltpu.SemaphoreType.DMA((2,)),
            pltpu.SemaphoreType.DMA((2,)),
        ],
        compiler_params=pltpu.CompilerParams(needs_layout_passes=False),
    )
    return f(input_ids, w).reshape(B, S, D)


def kernel(input_ids, W):
    return _run(input_ids, W)


# speedup vs baseline: 12.3718x; 10.8453x over previous
import jax
import jax.numpy as jnp
from jax import lax
from jax.experimental import pallas as pl
from jax.experimental.pallas import tpu as pltpu
from jax.experimental.pallas import tpu_sc as plsc

B, S, D = 4, 4096, 2048
NC, NS, L = 2, 16, 16
NW = NC * NS
CHUNK = (B * S) // NW
CPR = S // CHUNK
C = 16
G = 16


def _wid():
    return lax.axis_index("s") * NC + lax.axis_index("c")


def _body(ids_hbm, w_hbm, out_hbm, row_v, cache_v, sem):
    wid = _wid()
    r = wid // CPR
    k = wid % CPR
    base = k * CHUNK

    pltpu.sync_copy(ids_hbm.at[r], row_v)
    pltpu.sync_copy(w_hbm.at[pl.ds(0, C)], cache_v)

    lane = lax.iota(jnp.int32, 16)

    def prefix_step(j, carry):
        ids = row_v[pl.ds(j * L, L)]
        nd = jnp.where(ids < 10, jnp.int32(-1), lane + j * L)
        return jnp.maximum(carry, jnp.max(nd))

    carry0 = lax.fori_loop(0, base // L, prefix_step, jnp.int32(-1))

    out_base = wid * CHUNK

    def grp_step(g, ln):
        v = row_v[pl.ds(base + g * L, L)]
        for t in range(L):
            i = base + g * L + t
            digit = v[t] < 10
            ln = jnp.where(digit, ln, i)
            pos = i - ln

            @pl.when(pos < C)
            def _(pos=pos, i=i):
                pltpu.async_copy(
                    cache_v.at[pos], out_hbm.at[out_base - base + i], sem
                )

            @pl.when(pos >= C)
            def _(pos=pos, i=i):
                pltpu.async_copy(
                    w_hbm.at[pos], out_hbm.at[out_base - base + i], sem
                )

        return ln

    lax.fori_loop(0, CHUNK // L, grp_step, carry0)

    def drain_step(j, c):
        pltpu.make_async_copy(w_hbm.at[pl.ds(0, G)], cache_v, sem).wait()
        return c

    lax.fori_loop(0, CHUNK // G, drain_step, 0)


@jax.jit
def _run(input_ids, w):
    mesh = plsc.VectorSubcoreMesh(
        core_axis_name="c", subcore_axis_name="s", num_cores=NC, num_subcores=NS
    )
    f = pl.kernel(
        _body,
        out_type=jax.ShapeDtypeStruct((B * S, D), jnp.float32),
        mesh=mesh,
        scratch_types=[
            pltpu.VMEM((S,), jnp.int32),
            pltpu.VMEM((C, D), jnp.float32),
            pltpu.SemaphoreType.DMA,
        ],
        compiler_params=pltpu.CompilerParams(needs_layout_passes=False),
    )
    return f(input_ids, w).reshape(B, S, D)


def kernel(input_ids, W):
    return _run(input_ids, W)
